# baseline (speedup 1.0000x reference)
import functools

import jax
import jax.numpy as jnp
from jax import lax
from jax.experimental import pallas as pl
from jax.experimental.pallas import tpu as pltpu
from jax.experimental.pallas import tpu_sc as plsc

_N = 10000
_NPAD = 10240
_E = 320000
_EPAD = 344064
_H = 16
_NW = 32
_ROWS_W = _NPAD // _NW
_ROWS_T = _NPAD // 16
_GPW = _EPAD // _NW // 128

_SC_PARAMS = pltpu.CompilerParams(use_tc_tiling_on_sc=False,
                                  needs_layout_passes=False)


@functools.cache
def _sc_mesh():
    return plsc.VectorSubcoreMesh(core_axis_name="c", subcore_axis_name="s")


def _feats_body(x_ref, w_ref, a1_ref, a2_ref, h_ref, s_ref):
    xb = x_ref[...]
    nh = w_ref.shape[0]
    hs = []
    cols = []
    for i in range(nh):
        h = jnp.dot(xb, w_ref[i], preferred_element_type=jnp.float32)
        hs.append(h)
        cols.append(jnp.sum(h * a1_ref[i][None, :], axis=1))
        cols.append(jnp.sum(h * a2_ref[i][None, :], axis=1))
    h_ref[...] = jnp.concatenate(hs, axis=1) if nh > 1 else hs[0]
    s_ref[...] = jnp.stack(cols, axis=1)


def _tc_feats(xp, w, a1, a2):
    nh, f = w.shape[0], w.shape[1]
    blk = 1024
    hcat, s = pl.pallas_call(
        _feats_body,
        grid=(_NPAD // blk,),
        in_specs=[pl.BlockSpec((blk, f), lambda i: (i, 0)),
                  pl.BlockSpec((nh, f, _H), lambda i: (0, 0, 0)),
                  pl.BlockSpec((nh, _H), lambda i: (0, 0)),
                  pl.BlockSpec((nh, _H), lambda i: (0, 0))],
        out_specs=[pl.BlockSpec((blk, nh * _H), lambda i: (i, 0)),
                   pl.BlockSpec((blk, 2 * nh), lambda i: (i, 0))],
        out_shape=[jax.ShapeDtypeStruct((_NPAD, nh * _H), jnp.float32),
                   jax.ShapeDtypeStruct((_NPAD, 2 * nh), jnp.float32)],
    )(xp, w, a1, a2)
    return hcat, s


def _lsm_body(o_ref, out_ref):
    o = o_ref[...]
    m = jnp.max(o, axis=1, keepdims=True)
    ex = jnp.exp(o - m)
    lse = jnp.log(jnp.sum(ex, axis=1, keepdims=True)) + m
    out_ref[...] = o - lse


def _tc_log_softmax(o2):
    blk = 1024
    return pl.pallas_call(
        _lsm_body,
        grid=(_NPAD // blk,),
        in_specs=[pl.BlockSpec((blk, _H), lambda i: (i, 0))],
        out_specs=pl.BlockSpec((blk, _H), lambda i: (i, 0)),
        out_shape=jax.ShapeDtypeStruct((_NPAD, _H), jnp.float32),
    )(o2)


def _edge_body(nh, h_hbm, s_hbm, eidx_hbm,
               num_hbm, den_hbm,
               s_loc, eb, rowsg, packed, exb, dstb, zrow, zden, num_sh, den_sh,
               semi0, semi1, semi2, semg0, semg1, semg2,
               semn0, semn1, semd0, semd1):
    hw = nh * _H
    cid = lax.axis_index("c")
    sid = lax.axis_index("s")
    wid = cid * 16 + sid
    zv = jnp.zeros((16,), jnp.float32)
    semi = (semi0, semi1, semi2)
    semg = (semg0, semg1, semg2)
    semn = (semn0, semn1)
    semd = (semd0, semd1)

    def zrow_f(i, c):
        for q in range(hw // 16):
            zrow[i, pl.ds(16 * q, 16)] = zv
        return c
    lax.fori_loop(0, 40, zrow_f, 0)

    def zden_f(i, c):
        zden[pl.ds(16 * i, 16)] = zv
        return c
    lax.fori_loop(0, _ROWS_T // 16, zden_f, 0)
    for rep in range(_ROWS_T // 40):
        pltpu.sync_copy(zrow, num_sh.at[pl.ds(sid * _ROWS_T + rep * 40, 40)])
    for i in range(nh):
        pltpu.sync_copy(zden, den_sh.at[i, pl.ds(sid * _ROWS_T, _ROWS_T)])

    pltpu.sync_copy(s_hbm, s_loc)
    plsc.subcore_barrier()

    g0 = wid * _GPW

    def idx_issue(g, b):
        pltpu.async_copy(eidx_hbm.at[g0 + g], eb.at[b], semi[b])

    def idx_wait(b):
        pltpu.make_async_copy(eidx_hbm.at[g0], eb.at[b], semi[b]).wait()

    def gather_issue(b):
        pltpu.async_copy(h_hbm.at[eb.at[b, 0]], rowsg.at[b], semg[b])

    def gather_wait(b):
        pltpu.make_async_copy(h_hbm.at[eb.at[b, 0]], rowsg.at[b],
                              semg[b]).wait()

    def scat_issue(b):
        pltpu.async_copy(packed.at[b], num_sh.at[dstb.at[b]], semn[b],
                         add=True)
        for i in range(nh):
            pltpu.async_copy(exb.at[b, i], den_sh.at[i].at[dstb.at[b]],
                             semd[b], add=True)

    def scat_wait(b):
        pltpu.make_async_copy(packed.at[b], num_sh.at[dstb.at[b]],
                              semn[b]).wait()
        for i in range(nh):
            pltpu.make_async_copy(exb.at[b, i], den_sh.at[i].at[dstb.at[b]],
                                  semd[b]).wait()

    idx_issue(0, 0)
    idx_issue(1, 1)
    idx_wait(0)
    gather_issue(0)
    idx_wait(1)
    gather_issue(1)
    idx_issue(2, 2)

    def six(k, c):
        for off in range(6):
            g = 6 * k + off
            bg = off % 3
            bi = (off + 2) % 3
            bs = off % 2
            idx_wait(bi)
            gather_issue(bi)
            gather_wait(bg)

            @pl.when(g >= 2)
            def _():
                scat_wait(bs)
            for m in range(8):
                sv = eb[bg, 0, pl.ds(m * 16, 16)]
                dv = eb[bg, 1, pl.ds(m * 16, 16)]
                av = plsc.bitcast(eb[bg, 2, pl.ds(m * 16, 16)], jnp.float32)
                dstb[bs, pl.ds(m * 16, 16)] = dv
                wvs = []
                for i in range(nh):
                    e = (plsc.load_gather(s_loc.at[2 * i], [sv])
                         + plsc.load_gather(s_loc.at[2 * i + 1], [dv]))
                    e = jnp.where(e >= 0.0, e, 0.2 * e)
                    ex = jnp.exp(e)
                    exb[bs, i, pl.ds(m * 16, 16)] = ex
                    wvs.append(ex * av)
                for j in range(16):
                    r = m * 16 + j
                    for i in range(nh):
                        packed[bs, r, pl.ds(16 * i, 16)] = (
                            rowsg[bg, r, pl.ds(16 * i, 16)] * wvs[i][j])
            idx_issue(g + 3, bg)
            scat_issue(bs)
        return c
    lax.fori_loop(0, _GPW // 6, six, 0)
    gather_wait(0)
    gather_wait(1)
    idx_wait(2)
    scat_wait(0)
    scat_wait(1)

    plsc.subcore_barrier()
    pltpu.sync_copy(num_sh.at[pl.ds(sid * _ROWS_T, _ROWS_T)],
                    num_hbm.at[cid, pl.ds(sid * _ROWS_T, _ROWS_T)])
    for i in range(nh):
        pltpu.sync_copy(den_sh.at[i, pl.ds(sid * _ROWS_T, _ROWS_T)],
                        den_hbm.at[cid, i, pl.ds(sid * _ROWS_T, _ROWS_T)])


@functools.cache
def _edge_call(nh):
    hw = nh * _H
    return pl.kernel(
        functools.partial(_edge_body, nh),
        compiler_params=_SC_PARAMS,
        out_type=[jax.ShapeDtypeStruct((2, _NPAD, hw), jnp.float32),
                  jax.ShapeDtypeStruct((2, nh, _NPAD), jnp.float32)],
        mesh=_sc_mesh(),
        scratch_types=[
            pltpu.VMEM((2 * nh, _NPAD), jnp.float32),
            pltpu.VMEM((3, 3, 128), jnp.int32),
            pltpu.VMEM((3, 128, hw), jnp.float32),
            pltpu.VMEM((2, 128, hw), jnp.float32),
            pltpu.VMEM((2, nh, 128), jnp.float32),
            pltpu.VMEM((2, 128), jnp.int32),
            pltpu.VMEM((40, hw), jnp.float32),
            pltpu.VMEM((_ROWS_T,), jnp.float32),
            pltpu.VMEM_SHARED((_NPAD, hw), jnp.float32),
            pltpu.VMEM_SHARED((nh, _NPAD), jnp.float32),
        ] + [pltpu.SemaphoreType.DMA] * 10,
    )


def _node_body(nh, num_hbm, den_hbm, d_hbm, gemb_hbm, bemb_hbm,
               out_hbm, film_hbm,
               numl, numl2, denl, denl2, didx, gam, bet, outl, fbuf, sem,
               sem2):
    hw = nh * _H
    cid = lax.axis_index("c")
    sid = lax.axis_index("s")
    wid = cid * 16 + sid
    base = wid * _ROWS_W
    zv = jnp.zeros((16,), jnp.float32)

    pends = [pltpu.async_copy(num_hbm.at[0, pl.ds(base, _ROWS_W)], numl, sem),
             pltpu.async_copy(num_hbm.at[1, pl.ds(base, _ROWS_W)], numl2,
                              sem)]
    for i in range(nh):
        pends.append(pltpu.async_copy(den_hbm.at[0, i, pl.ds(base, _ROWS_W)],
                                      denl.at[i], sem))
        pends.append(pltpu.async_copy(den_hbm.at[1, i, pl.ds(base, _ROWS_W)],
                                      denl2.at[i], sem))
    dpends = [pltpu.async_copy(d_hbm.at[pl.ds(base + 80 * j, 80)], didx.at[j],
                               sem2) for j in range(4)]
    for p in dpends:
        p.wait()
    for i in range(nh):
        for j in range(4):
            pends.append(pltpu.async_copy(gemb_hbm.at[i].at[didx.at[j]],
                                          gam.at[i, pl.ds(80 * j, 80)], sem))
            pends.append(pltpu.async_copy(bemb_hbm.at[i].at[didx.at[j]],
                                          bet.at[i, pl.ds(80 * j, 80)], sem))
    for p in pends:
        p.wait()

    def blockf(t, carry):
        accs = list(carry)
        denvs = [denl[i, pl.ds(16 * t, 16)] + denl2[i, pl.ds(16 * t, 16)]
                 for i in range(nh)]
        for j in range(16):
            r = 16 * t + j
            for i in range(nh):
                nv = numl[r, pl.ds(16 * i, 16)] + numl2[r, pl.ds(16 * i, 16)]
                tt = nv / (denvs[i][j] + 1e-16)
                g = gam[i, r]
                b = bet[i, r]
                outl[r, pl.ds(16 * i, 16)] = g * tt + b
                accs[2 * i] = accs[2 * i] + g * g
                accs[2 * i + 1] = accs[2 * i + 1] + b * b
        return tuple(accs)
    accs = lax.fori_loop(0, _ROWS_W // 16, blockf, (zv,) * (2 * nh))
    for i in range(2 * nh):
        fbuf[i] = accs[i]
    pltpu.sync_copy(outl, out_hbm.at[pl.ds(base, _ROWS_W)])
    pltpu.sync_copy(fbuf, film_hbm.at[wid])


@functools.cache
def _node_call(nh):
    hw = nh * _H
    return pl.kernel(
        functools.partial(_node_body, nh),
        compiler_params=_SC_PARAMS,
        out_type=[jax.ShapeDtypeStruct((_NPAD, hw), jnp.float32),
                  jax.ShapeDtypeStruct((_NW, 2 * nh, 16), jnp.float32)],
        mesh=_sc_mesh(),
        scratch_types=[
            pltpu.VMEM((_ROWS_W, hw), jnp.float32),
            pltpu.VMEM((_ROWS_W, hw), jnp.float32),
            pltpu.VMEM((nh, _ROWS_W), jnp.float32),
            pltpu.VMEM((nh, _ROWS_W), jnp.float32),
            pltpu.VMEM((4, 80), jnp.int32),
            pltpu.VMEM((nh, _ROWS_W, _H), jnp.float32),
            pltpu.VMEM((nh, _ROWS_W, _H), jnp.float32),
            pltpu.VMEM((_ROWS_W, hw), jnp.float32),
            pltpu.VMEM((2 * nh, 16), jnp.float32),
            pltpu.SemaphoreType.DMA,
            pltpu.SemaphoreType.DMA,
        ],
    )


def _bias_body(idx_hbm, xc_hbm, o2_hbm, parts_hbm, idxl, rows48, rows16,
               fbuf, sem):
    cid = lax.axis_index("c")
    sid = lax.axis_index("s")
    wid = cid * 16 + sid
    zv = jnp.zeros((16,), jnp.float32)
    pltpu.sync_copy(idx_hbm.at[wid], idxl)
    pltpu.async_copy(xc_hbm.at[idxl], rows48, sem).wait()
    pltpu.async_copy(o2_hbm.at[idxl], rows16, sem).wait()

    def f(r, accs):
        a0, a1, a2, a3 = accs
        v0 = rows48[r, pl.ds(0, 16)]
        v1 = rows48[r, pl.ds(16, 16)]
        v2 = rows48[r, pl.ds(32, 16)]
        v3 = rows16[r]
        return (a0 + v0 * v0, a1 + v1 * v1, a2 + v2 * v2, a3 + v3 * v3)
    accs = lax.fori_loop(0, 32, f, (zv, zv, zv, zv))
    for i in range(4):
        fbuf[i] = accs[i]
    pltpu.sync_copy(fbuf, parts_hbm.at[wid])


@functools.cache
def _bias_call():
    return pl.kernel(
        _bias_body,
        compiler_params=_SC_PARAMS,
        out_type=jax.ShapeDtypeStruct((_NW, 4, 16), jnp.float32),
        mesh=_sc_mesh(),
        scratch_types=[
            pltpu.VMEM((32,), jnp.int32),
            pltpu.VMEM((32, 3 * _H), jnp.float32),
            pltpu.VMEM((32, _H), jnp.float32),
            pltpu.VMEM((4, 16), jnp.float32),
            pltpu.SemaphoreType.DMA,
        ],
    )


def kernel(x, adj, d, idx, edge, W_heads, a1_heads, a2_heads, gemb_heads,
           bemb_heads, W_out, a1_out, a2_out, gemb_out, bemb_out):
    x_pad = jnp.pad(x, ((0, _NPAD - _N), (0, 0)))
    src_p = jnp.pad(edge[0].astype(jnp.int32),
                    (0, _EPAD - _E)).reshape(_EPAD // 128, 128)
    dst_p = jnp.pad(edge[1].astype(jnp.int32), (0, _EPAD - _E),
                    constant_values=_NPAD - 1).reshape(_EPAD // 128, 128)
    adj_p = lax.bitcast_convert_type(
        jnp.pad(adj, (0, _EPAD - _E)), jnp.int32).reshape(_EPAD // 128, 128)
    eidx = jnp.pad(jnp.stack([src_p, dst_p, adj_p], axis=1),
                   ((0, 3), (0, 0), (0, 0)))
    zr = gemb_heads.shape[1]
    d_pad = jnp.pad(d.astype(jnp.int32), (0, _NPAD - _N), constant_values=zr)
    gemb_h = jnp.pad(gemb_heads, ((0, 0), (0, 1), (0, 0)))
    bemb_h = jnp.pad(bemb_heads, ((0, 0), (0, 1), (0, 0)))
    gemb_o = jnp.pad(gemb_out, ((0, 1), (0, 0)))[None]
    bemb_o = jnp.pad(bemb_out, ((0, 1), (0, 0)))[None]
    idx_p = jnp.pad(idx.astype(jnp.int32), (0, 1024 - 1000),
                    constant_values=_NPAD - 1).reshape(_NW, 32)

    hcat, s6 = _tc_feats(x_pad, W_heads, a1_heads, a2_heads)
    num1, den1 = _edge_call(3)(hcat, s6.T, eidx)
    xcat, film1 = _node_call(3)(num1, den1, d_pad, gemb_h, bemb_h)

    h2, s2 = _tc_feats(xcat, W_out[None], a1_out[None], a2_out[None])
    num2, den2 = _edge_call(1)(h2, s2.T, eidx)
    o2, film2 = _node_call(1)(num2, den2, d_pad, gemb_o, bemb_o)

    logp = _tc_log_softmax(o2)[:_N]
    parts = _bias_call()(idx_p, xcat, o2)

    scale_f = 1.0 / (_N * _H)
    f1 = jnp.sum(film1) * (scale_f / 3.0)
    f2 = jnp.sum(film2) * scale_f
    scale_b = 1.0 / (1000 * _H)
    b1 = jnp.sum(parts[:, :3]) * (scale_b / 3.0)
    b2 = jnp.sum(parts[:, 3]) * scale_b
    return (logp, b1 + b2, f1 + f2)


# speedup vs baseline: 2.0686x; 2.0686x over previous
import functools

import jax
import jax.numpy as jnp
from jax import lax
from jax.experimental import pallas as pl
from jax.experimental.pallas import tpu as pltpu
from jax.experimental.pallas import tpu_sc as plsc

_N = 10000
_NPAD = 10240
_E = 320000
_EPAD = 327680
_H = 16
_NW = 32
_ROWS_W = _NPAD // _NW
_ROWS_T = _NPAD // 16
_GPW = _EPAD // _NW // 128

_SC_PARAMS = pltpu.CompilerParams(use_tc_tiling_on_sc=False,
                                  needs_layout_passes=False)


@functools.cache
def _sc_mesh():
    return plsc.VectorSubcoreMesh(core_axis_name="c", subcore_axis_name="s")


def _feats_body(x_ref, w_ref, a1_ref, a2_ref, h_ref, s_ref):
    xb = x_ref[...]
    nh = w_ref.shape[0]
    hs = []
    cols = []
    for i in range(nh):
        h = jnp.dot(xb, w_ref[i], preferred_element_type=jnp.float32)
        hs.append(h)
        cols.append(jnp.sum(h * a1_ref[i][None, :], axis=1))
        cols.append(jnp.sum(h * a2_ref[i][None, :], axis=1))
    h_ref[...] = jnp.concatenate(hs, axis=1) if nh > 1 else hs[0]
    s_ref[...] = jnp.stack(cols, axis=1)


def _tc_feats(xp, w, a1, a2):
    nh, f = w.shape[0], w.shape[1]
    blk = 1024
    hcat, s = pl.pallas_call(
        _feats_body,
        grid=(_NPAD // blk,),
        in_specs=[pl.BlockSpec((blk, f), lambda i: (i, 0)),
                  pl.BlockSpec((nh, f, _H), lambda i: (0, 0, 0)),
                  pl.BlockSpec((nh, _H), lambda i: (0, 0)),
                  pl.BlockSpec((nh, _H), lambda i: (0, 0))],
        out_specs=[pl.BlockSpec((blk, nh * _H), lambda i: (i, 0)),
                   pl.BlockSpec((blk, 2 * nh), lambda i: (i, 0))],
        out_shape=[jax.ShapeDtypeStruct((_NPAD, nh * _H), jnp.float32),
                   jax.ShapeDtypeStruct((_NPAD, 2 * nh), jnp.float32)],
    )(xp, w, a1, a2)
    return hcat, s


def _lsm_body(o_ref, out_ref):
    o = o_ref[...]
    m = jnp.max(o, axis=1, keepdims=True)
    ex = jnp.exp(o - m)
    lse = jnp.log(jnp.sum(ex, axis=1, keepdims=True)) + m
    out_ref[...] = o - lse


def _tc_log_softmax(o2):
    blk = 1024
    return pl.pallas_call(
        _lsm_body,
        grid=(_NPAD // blk,),
        in_specs=[pl.BlockSpec((blk, _H), lambda i: (i, 0))],
        out_specs=pl.BlockSpec((blk, _H), lambda i: (i, 0)),
        out_shape=jax.ShapeDtypeStruct((_NPAD, _H), jnp.float32),
    )(o2)


def _edge_body(nh, h_hbm, s_hbm, eidx_hbm,
               num_hbm, den_hbm,
               s_loc, eb, rowsg, packed, exb, dstb, zrow, zden, num_sh, den_sh,
               semi0, semi1, semg0, semg1, semg2a, semg2b,
               semn0, semn1, semd0, semd1):
    hw = nh * _H
    cid = lax.axis_index("c")
    sid = lax.axis_index("s")
    wid = cid * 16 + sid
    zv = jnp.zeros((16,), jnp.float32)
    semi = (semi0, semi1)
    semg = (semg0, semg1)
    semg2 = (semg2a, semg2b)
    semn = (semn0, semn1)
    semd = (semd0, semd1)

    def zrow_f(i, c):
        for q in range(hw // 16):
            zrow[i, pl.ds(16 * q, 16)] = zv
        return c
    lax.fori_loop(0, 80, zrow_f, 0)

    def zden_f(i, c):
        zden[pl.ds(16 * i, 16)] = zv
        return c
    lax.fori_loop(0, _ROWS_T // 16, zden_f, 0)
    for rep in range(_ROWS_T // 80):
        pltpu.sync_copy(zrow, num_sh.at[pl.ds(sid * _ROWS_T + rep * 80, 80)])
    for i in range(nh):
        pltpu.sync_copy(zden, den_sh.at[i, pl.ds(sid * _ROWS_T, _ROWS_T)])

    pltpu.sync_copy(s_hbm, s_loc)
    plsc.subcore_barrier()

    g0 = wid * _GPW

    def idx_issue(g, b):
        pltpu.async_copy(eidx_hbm.at[g0 + g], eb.at[b], semi[b])

    def idx_wait(b):
        pltpu.make_async_copy(eidx_hbm.at[g0], eb.at[b], semi[b]).wait()

    def gather_issue(b):
        pltpu.async_copy(h_hbm.at[eb.at[b, 0].at[pl.ds(0, 64)]],
                         rowsg.at[b, pl.ds(0, 64)], semg[b])
        pltpu.async_copy(h_hbm.at[eb.at[b, 0].at[pl.ds(64, 64)]],
                         rowsg.at[b, pl.ds(64, 64)], semg2[b])

    def gather_wait(b):
        pltpu.make_async_copy(h_hbm.at[eb.at[b, 0].at[pl.ds(0, 64)]],
                              rowsg.at[b, pl.ds(0, 64)], semg[b]).wait()
        pltpu.make_async_copy(h_hbm.at[eb.at[b, 0].at[pl.ds(64, 64)]],
                              rowsg.at[b, pl.ds(64, 64)], semg2[b]).wait()

    def scat_issue(b):
        pltpu.async_copy(packed.at[b], num_sh.at[dstb.at[b]], semn[b],
                         add=True)
        for i in range(nh):
            pltpu.async_copy(exb.at[b, i], den_sh.at[i].at[dstb.at[b]],
                             semd[b], add=True)

    def scat_wait(b):
        pltpu.make_async_copy(packed.at[b], num_sh.at[dstb.at[b]],
                              semn[b]).wait()
        for i in range(nh):
            pltpu.make_async_copy(exb.at[b, i], den_sh.at[i].at[dstb.at[b]],
                                  semd[b]).wait()

    pltpu.sync_copy(eidx_hbm.at[g0], eb.at[0])
    gather_issue(0)
    idx_issue(1, 1)

    def pair(k, c):
        for b in range(2):
            g = 2 * k + b
            b2 = 1 - b
            idx_wait(b2)
            gather_issue(b2)
            gather_wait(b)

            @pl.when(g >= 2)
            def _():
                scat_wait(b)
            for m in range(8):
                sv = eb[b, 0, pl.ds(m * 16, 16)]
                dv = eb[b, 1, pl.ds(m * 16, 16)]
                av = plsc.bitcast(eb[b, 2, pl.ds(m * 16, 16)], jnp.float32)
                dstb[b, pl.ds(m * 16, 16)] = dv
                wvs = []
                for i in range(nh):
                    e = (plsc.load_gather(s_loc.at[2 * i], [sv])
                         + plsc.load_gather(s_loc.at[2 * i + 1], [dv]))
                    e = jnp.where(e >= 0.0, e, 0.2 * e)
                    ex = jnp.exp(e)
                    exb[b, i, pl.ds(m * 16, 16)] = ex
                    wvs.append(ex * av)
                for j in range(16):
                    r = m * 16 + j
                    for i in range(nh):
                        packed[b, r, pl.ds(16 * i, 16)] = (
                            rowsg[b, r, pl.ds(16 * i, 16)] * wvs[i][j])
            idx_issue(g + 2, b)
            scat_issue(b)
        return c
    lax.fori_loop(0, _GPW // 2, pair, 0)
    gather_wait(0)
    idx_wait(1)
    scat_wait(0)
    scat_wait(1)

    plsc.subcore_barrier()
    pltpu.sync_copy(num_sh.at[pl.ds(sid * _ROWS_T, _ROWS_T)],
                    num_hbm.at[cid, pl.ds(sid * _ROWS_T, _ROWS_T)])
    for i in range(nh):
        pltpu.sync_copy(den_sh.at[i, pl.ds(sid * _ROWS_T, _ROWS_T)],
                        den_hbm.at[cid, i, pl.ds(sid * _ROWS_T, _ROWS_T)])


@functools.cache
def _edge_call(nh):
    hw = nh * _H
    return pl.kernel(
        functools.partial(_edge_body, nh),
        compiler_params=_SC_PARAMS,
        out_type=[jax.ShapeDtypeStruct((2, _NPAD, hw), jnp.float32),
                  jax.ShapeDtypeStruct((2, nh, _NPAD), jnp.float32)],
        mesh=_sc_mesh(),
        scratch_types=[
            pltpu.VMEM((2 * nh, _NPAD), jnp.float32),
            pltpu.VMEM((2, 3, 128), jnp.int32),
            pltpu.VMEM((2, 128, hw), jnp.float32),
            pltpu.VMEM((2, 128, hw), jnp.float32),
            pltpu.VMEM((2, nh, 128), jnp.float32),
            pltpu.VMEM((2, 128), jnp.int32),
            pltpu.VMEM((80, hw), jnp.float32),
            pltpu.VMEM((_ROWS_T,), jnp.float32),
            pltpu.VMEM_SHARED((_NPAD, hw), jnp.float32),
            pltpu.VMEM_SHARED((nh, _NPAD), jnp.float32),
        ] + [pltpu.SemaphoreType.DMA] * 10,
    )


def _node_body(nh, num_hbm, den_hbm, d_hbm, gemb_hbm, bemb_hbm,
               out_hbm, film_hbm,
               numl, numl2, denl, denl2, didx, gam, bet, outl, fbuf, sem,
               sem2):
    hw = nh * _H
    cid = lax.axis_index("c")
    sid = lax.axis_index("s")
    wid = cid * 16 + sid
    base = wid * _ROWS_W
    zv = jnp.zeros((16,), jnp.float32)

    pends = [pltpu.async_copy(num_hbm.at[0, pl.ds(base, _ROWS_W)], numl, sem),
             pltpu.async_copy(num_hbm.at[1, pl.ds(base, _ROWS_W)], numl2,
                              sem)]
    for i in range(nh):
        pends.append(pltpu.async_copy(den_hbm.at[0, i, pl.ds(base, _ROWS_W)],
                                      denl.at[i], sem))
        pends.append(pltpu.async_copy(den_hbm.at[1, i, pl.ds(base, _ROWS_W)],
                                      denl2.at[i], sem))
    dpends = [pltpu.async_copy(d_hbm.at[pl.ds(base + 80 * j, 80)], didx.at[j],
                               sem2) for j in range(4)]
    for p in dpends:
        p.wait()
    for i in range(nh):
        for j in range(4):
            pends.append(pltpu.async_copy(gemb_hbm.at[i].at[didx.at[j]],
                                          gam.at[i, pl.ds(80 * j, 80)], sem))
            pends.append(pltpu.async_copy(bemb_hbm.at[i].at[didx.at[j]],
                                          bet.at[i, pl.ds(80 * j, 80)], sem))
    for p in pends:
        p.wait()

    def blockf(t, carry):
        accs = list(carry)
        denvs = [denl[i, pl.ds(16 * t, 16)] + denl2[i, pl.ds(16 * t, 16)]
                 for i in range(nh)]
        for j in range(16):
            r = 16 * t + j
            for i in range(nh):
                nv = numl[r, pl.ds(16 * i, 16)] + numl2[r, pl.ds(16 * i, 16)]
                tt = nv / (denvs[i][j] + 1e-16)
                g = gam[i, r]
                b = bet[i, r]
                outl[r, pl.ds(16 * i, 16)] = g * tt + b
                accs[2 * i] = accs[2 * i] + g * g
                accs[2 * i + 1] = accs[2 * i + 1] + b * b
        return tuple(accs)
    accs = lax.fori_loop(0, _ROWS_W // 16, blockf, (zv,) * (2 * nh))
    for i in range(2 * nh):
        fbuf[i] = accs[i]
    pltpu.sync_copy(outl, out_hbm.at[pl.ds(base, _ROWS_W)])
    pltpu.sync_copy(fbuf, film_hbm.at[wid])


@functools.cache
def _node_call(nh):
    hw = nh * _H
    return pl.kernel(
        functools.partial(_node_body, nh),
        compiler_params=_SC_PARAMS,
        out_type=[jax.ShapeDtypeStruct((_NPAD, hw), jnp.float32),
                  jax.ShapeDtypeStruct((_NW, 2 * nh, 16), jnp.float32)],
        mesh=_sc_mesh(),
        scratch_types=[
            pltpu.VMEM((_ROWS_W, hw), jnp.float32),
            pltpu.VMEM((_ROWS_W, hw), jnp.float32),
            pltpu.VMEM((nh, _ROWS_W), jnp.float32),
            pltpu.VMEM((nh, _ROWS_W), jnp.float32),
            pltpu.VMEM((4, 80), jnp.int32),
            pltpu.VMEM((nh, _ROWS_W, _H), jnp.float32),
            pltpu.VMEM((nh, _ROWS_W, _H), jnp.float32),
            pltpu.VMEM((_ROWS_W, hw), jnp.float32),
            pltpu.VMEM((2 * nh, 16), jnp.float32),
            pltpu.SemaphoreType.DMA,
            pltpu.SemaphoreType.DMA,
        ],
    )


def _bias_body(idx_hbm, xc_hbm, o2_hbm, parts_hbm, idxl, rows48, rows16,
               fbuf, sem):
    cid = lax.axis_index("c")
    sid = lax.axis_index("s")
    wid = cid * 16 + sid
    zv = jnp.zeros((16,), jnp.float32)
    pltpu.sync_copy(idx_hbm.at[wid], idxl)
    pltpu.async_copy(xc_hbm.at[idxl], rows48, sem).wait()
    pltpu.async_copy(o2_hbm.at[idxl], rows16, sem).wait()

    def f(r, accs):
        a0, a1, a2, a3 = accs
        v0 = rows48[r, pl.ds(0, 16)]
        v1 = rows48[r, pl.ds(16, 16)]
        v2 = rows48[r, pl.ds(32, 16)]
        v3 = rows16[r]
        return (a0 + v0 * v0, a1 + v1 * v1, a2 + v2 * v2, a3 + v3 * v3)
    accs = lax.fori_loop(0, 32, f, (zv, zv, zv, zv))
    for i in range(4):
        fbuf[i] = accs[i]
    pltpu.sync_copy(fbuf, parts_hbm.at[wid])


@functools.cache
def _bias_call():
    return pl.kernel(
        _bias_body,
        compiler_params=_SC_PARAMS,
        out_type=jax.ShapeDtypeStruct((_NW, 4, 16), jnp.float32),
        mesh=_sc_mesh(),
        scratch_types=[
            pltpu.VMEM((32,), jnp.int32),
            pltpu.VMEM((32, 3 * _H), jnp.float32),
            pltpu.VMEM((32, _H), jnp.float32),
            pltpu.VMEM((4, 16), jnp.float32),
            pltpu.SemaphoreType.DMA,
        ],
    )


def kernel(x, adj, d, idx, edge, W_heads, a1_heads, a2_heads, gemb_heads,
           bemb_heads, W_out, a1_out, a2_out, gemb_out, bemb_out):
    x_pad = jnp.pad(x, ((0, _NPAD - _N), (0, 0)))
    src_p = jnp.pad(edge[0].astype(jnp.int32),
                    (0, _EPAD - _E)).reshape(_EPAD // 128, 128)
    dst_p = jnp.pad(edge[1].astype(jnp.int32), (0, _EPAD - _E),
                    constant_values=_NPAD - 1).reshape(_EPAD // 128, 128)
    adj_p = lax.bitcast_convert_type(
        jnp.pad(adj, (0, _EPAD - _E)), jnp.int32).reshape(_EPAD // 128, 128)
    eidx = jnp.pad(jnp.stack([src_p, dst_p, adj_p], axis=1),
                   ((0, 2), (0, 0), (0, 0)))
    zr = gemb_heads.shape[1]
    d_pad = jnp.pad(d.astype(jnp.int32), (0, _NPAD - _N), constant_values=zr)
    gemb_h = jnp.pad(gemb_heads, ((0, 0), (0, 1), (0, 0)))
    bemb_h = jnp.pad(bemb_heads, ((0, 0), (0, 1), (0, 0)))
    gemb_o = jnp.pad(gemb_out, ((0, 1), (0, 0)))[None]
    bemb_o = jnp.pad(bemb_out, ((0, 1), (0, 0)))[None]
    idx_p = jnp.pad(idx.astype(jnp.int32), (0, 1024 - 1000),
                    constant_values=_NPAD - 1).reshape(_NW, 32)

    hcat, s6 = _tc_feats(x_pad, W_heads, a1_heads, a2_heads)
    num1, den1 = _edge_call(3)(hcat, s6.T, eidx)
    xcat, film1 = _node_call(3)(num1, den1, d_pad, gemb_h, bemb_h)

    h2, s2 = _tc_feats(xcat, W_out[None], a1_out[None], a2_out[None])
    num2, den2 = _edge_call(1)(h2, s2.T, eidx)
    o2, film2 = _node_call(1)(num2, den2, d_pad, gemb_o, bemb_o)

    logp = _tc_log_softmax(o2)[:_N]
    parts = _bias_call()(idx_p, xcat, o2)

    scale_f = 1.0 / (_N * _H)
    f1 = jnp.sum(film1) * (scale_f / 3.0)
    f2 = jnp.sum(film2) * scale_f
    scale_b = 1.0 / (1000 * _H)
    b1 = jnp.sum(parts[:, :3]) * (scale_b / 3.0)
    b2 = jnp.sum(parts[:, 3]) * scale_b
    return (logp, b1 + b2, f1 + f2)


# speedup vs baseline: 2.1730x; 1.0505x over previous
import functools

import jax
import jax.numpy as jnp
from jax import lax
from jax.experimental import pallas as pl
from jax.experimental.pallas import tpu as pltpu
from jax.experimental.pallas import tpu_sc as plsc

_N = 10000
_NPAD = 10240
_E = 320000
_EPAD = 327680
_H = 16
_NW = 32
_ROWS_W = _NPAD // _NW
_ROWS_T = _NPAD // 16
_GPW = _EPAD // _NW // 128
_SPLIT0 = 100

_SC_PARAMS = pltpu.CompilerParams(use_tc_tiling_on_sc=False,
                                  needs_layout_passes=False)


@functools.cache
def _sc_mesh():
    return plsc.VectorSubcoreMesh(core_axis_name="c", subcore_axis_name="s")


def _feats_body(x_ref, w_ref, a1_ref, a2_ref, h_ref, s_ref):
    xb = x_ref[...]
    nh = w_ref.shape[0]
    hs = []
    cols = []
    for i in range(nh):
        h = jnp.dot(xb, w_ref[i], preferred_element_type=jnp.float32)
        hs.append(h)
        cols.append(jnp.sum(h * a1_ref[i][None, :], axis=1))
        cols.append(jnp.sum(h * a2_ref[i][None, :], axis=1))
    h_ref[...] = jnp.concatenate(hs, axis=1) if nh > 1 else hs[0]
    s_ref[...] = jnp.stack(cols, axis=1)


def _tc_feats(xp, w, a1, a2):
    nh, f = w.shape[0], w.shape[1]
    blk = 1024
    hcat, s = pl.pallas_call(
        _feats_body,
        grid=(_NPAD // blk,),
        in_specs=[pl.BlockSpec((blk, f), lambda i: (i, 0)),
                  pl.BlockSpec((nh, f, _H), lambda i: (0, 0, 0)),
                  pl.BlockSpec((nh, _H), lambda i: (0, 0)),
                  pl.BlockSpec((nh, _H), lambda i: (0, 0))],
        out_specs=[pl.BlockSpec((blk, nh * _H), lambda i: (i, 0)),
                   pl.BlockSpec((blk, 2 * nh), lambda i: (i, 0))],
        out_shape=[jax.ShapeDtypeStruct((_NPAD, nh * _H), jnp.float32),
                   jax.ShapeDtypeStruct((_NPAD, 2 * nh), jnp.float32)],
    )(xp, w, a1, a2)
    return hcat, s


def _lsm_body(o_ref, out_ref):
    o = o_ref[...]
    m = jnp.max(o, axis=1, keepdims=True)
    ex = jnp.exp(o - m)
    lse = jnp.log(jnp.sum(ex, axis=1, keepdims=True)) + m
    out_ref[...] = o - lse


def _tc_log_softmax(o2):
    blk = 1024
    return pl.pallas_call(
        _lsm_body,
        grid=(_NPAD // blk,),
        in_specs=[pl.BlockSpec((blk, _H), lambda i: (i, 0))],
        out_specs=pl.BlockSpec((blk, _H), lambda i: (i, 0)),
        out_shape=jax.ShapeDtypeStruct((_NPAD, _H), jnp.float32),
    )(o2)


def _edge_body(nh, n0, n1, h_hbm, s_hbm, eidx_hbm,
               num_hbm, den_hbm,
               s_loc, eb, rowsg, packed, exb, dstb, zrow, zden, num_sh, den_sh,
               semi0, semi1, semg0, semg1, semg2a, semg2b,
               semn0, semn1, semd0, semd1):
    hw = nh * _H
    cid = lax.axis_index("c")
    sid = lax.axis_index("s")
    wid = cid * 16 + sid
    zv = jnp.zeros((16,), jnp.float32)
    semi = (semi0, semi1)
    semg = (semg0, semg1)
    semg2 = (semg2a, semg2b)
    semn = (semn0, semn1)
    semd = (semd0, semd1)

    def zrow_f(i, c):
        for q in range(hw // 16):
            zrow[i, pl.ds(16 * q, 16)] = zv
        return c
    lax.fori_loop(0, 80, zrow_f, 0)

    def zden_f(i, c):
        zden[pl.ds(16 * i, 16)] = zv
        return c
    lax.fori_loop(0, _ROWS_T // 16, zden_f, 0)
    for rep in range(_ROWS_T // 80):
        pltpu.sync_copy(zrow, num_sh.at[pl.ds(sid * _ROWS_T + rep * 80, 80)])
    for i in range(nh):
        pltpu.sync_copy(zden, den_sh.at[i, pl.ds(sid * _ROWS_T, _ROWS_T)])

    pltpu.sync_copy(s_hbm, s_loc)
    plsc.subcore_barrier()

    n_w = jnp.where(cid == 0, n0, n1)
    g0 = cid * 16 * n0 + sid * n_w
    pairs = jnp.where(cid == 0, n0 // 2, n1 // 2)

    def idx_issue(g, b):
        pltpu.async_copy(eidx_hbm.at[g0 + g], eb.at[b], semi[b])

    def idx_wait(b):
        pltpu.make_async_copy(eidx_hbm.at[g0], eb.at[b], semi[b]).wait()

    def gather_issue(b):
        pltpu.async_copy(h_hbm.at[eb.at[b, 0].at[pl.ds(0, 64)]],
                         rowsg.at[b, pl.ds(0, 64)], semg[b])
        pltpu.async_copy(h_hbm.at[eb.at[b, 0].at[pl.ds(64, 64)]],
                         rowsg.at[b, pl.ds(64, 64)], semg2[b])

    def gather_wait(b):
        pltpu.make_async_copy(h_hbm.at[eb.at[b, 0].at[pl.ds(0, 64)]],
                              rowsg.at[b, pl.ds(0, 64)], semg[b]).wait()
        pltpu.make_async_copy(h_hbm.at[eb.at[b, 0].at[pl.ds(64, 64)]],
                              rowsg.at[b, pl.ds(64, 64)], semg2[b]).wait()

    def scat_issue(b):
        pltpu.async_copy(packed.at[b], num_sh.at[dstb.at[b]], semn[b],
                         add=True)
        for i in range(nh):
            pltpu.async_copy(exb.at[b, i], den_sh.at[i].at[dstb.at[b]],
                             semd[b], add=True)

    def scat_wait(b):
        pltpu.make_async_copy(packed.at[b], num_sh.at[dstb.at[b]],
                              semn[b]).wait()
        for i in range(nh):
            pltpu.make_async_copy(exb.at[b, i], den_sh.at[i].at[dstb.at[b]],
                                  semd[b]).wait()

    pltpu.sync_copy(eidx_hbm.at[g0], eb.at[0])
    gather_issue(0)
    idx_issue(1, 1)

    def pair(k, c):
        for b in range(2):
            g = 2 * k + b
            b2 = 1 - b
            idx_wait(b2)
            gather_issue(b2)
            gather_wait(b)

            @pl.when(g >= 2)
            def _():
                scat_wait(b)
            for m in range(8):
                sv = eb[b, 0, pl.ds(m * 16, 16)]
                dv = eb[b, 1, pl.ds(m * 16, 16)]
                av = plsc.bitcast(eb[b, 2, pl.ds(m * 16, 16)], jnp.float32)
                dstb[b, pl.ds(m * 16, 16)] = dv
                wvs = []
                for i in range(nh):
                    e = (plsc.load_gather(s_loc.at[2 * i], [sv])
                         + plsc.load_gather(s_loc.at[2 * i + 1], [dv]))
                    e = jnp.where(e >= 0.0, e, 0.2 * e)
                    ex = jnp.exp(e)
                    exb[b, i, pl.ds(m * 16, 16)] = ex
                    wvs.append(ex * av)
                for j in range(16):
                    r = m * 16 + j
                    for i in range(nh):
                        packed[b, r, pl.ds(16 * i, 16)] = (
                            rowsg[b, r, pl.ds(16 * i, 16)] * wvs[i][j])
            idx_issue(g + 2, b)
            scat_issue(b)
        return c
    lax.fori_loop(0, pairs, pair, 0)
    gather_wait(0)
    idx_wait(1)
    scat_wait(0)
    scat_wait(1)

    plsc.subcore_barrier()
    pltpu.sync_copy(num_sh.at[pl.ds(sid * _ROWS_T, _ROWS_T)],
                    num_hbm.at[cid, pl.ds(sid * _ROWS_T, _ROWS_T)])
    for i in range(nh):
        pltpu.sync_copy(den_sh.at[i, pl.ds(sid * _ROWS_T, _ROWS_T)],
                        den_hbm.at[cid, i, pl.ds(sid * _ROWS_T, _ROWS_T)])


@functools.cache
def _edge_call(nh, n0=_GPW, n1=_GPW):
    hw = nh * _H
    return pl.kernel(
        functools.partial(_edge_body, nh, n0, n1),
        compiler_params=_SC_PARAMS,
        out_type=[jax.ShapeDtypeStruct((2, _NPAD, hw), jnp.float32),
                  jax.ShapeDtypeStruct((2, nh, _NPAD), jnp.float32)],
        mesh=_sc_mesh(),
        scratch_types=[
            pltpu.VMEM((2 * nh, _NPAD), jnp.float32),
            pltpu.VMEM((2, 3, 128), jnp.int32),
            pltpu.VMEM((2, 128, hw), jnp.float32),
            pltpu.VMEM((2, 128, hw), jnp.float32),
            pltpu.VMEM((2, nh, 128), jnp.float32),
            pltpu.VMEM((2, 128), jnp.int32),
            pltpu.VMEM((80, hw), jnp.float32),
            pltpu.VMEM((_ROWS_T,), jnp.float32),
            pltpu.VMEM_SHARED((_NPAD, hw), jnp.float32),
            pltpu.VMEM_SHARED((nh, _NPAD), jnp.float32),
        ] + [pltpu.SemaphoreType.DMA] * 10,
    )


def _node_body(nh, num_hbm, den_hbm, d_hbm, gemb_hbm, bemb_hbm,
               out_hbm, film_hbm,
               numl, numl2, denl, denl2, didx, gam, bet, outl, fbuf, sem,
               sem2):
    hw = nh * _H
    cid = lax.axis_index("c")
    sid = lax.axis_index("s")
    wid = cid * 16 + sid
    base = wid * _ROWS_W
    zv = jnp.zeros((16,), jnp.float32)

    pends = [pltpu.async_copy(num_hbm.at[0, pl.ds(base, _ROWS_W)], numl, sem),
             pltpu.async_copy(num_hbm.at[1, pl.ds(base, _ROWS_W)], numl2,
                              sem)]
    for i in range(nh):
        pends.append(pltpu.async_copy(den_hbm.at[0, i, pl.ds(base, _ROWS_W)],
                                      denl.at[i], sem))
        pends.append(pltpu.async_copy(den_hbm.at[1, i, pl.ds(base, _ROWS_W)],
                                      denl2.at[i], sem))
    dpends = [pltpu.async_copy(d_hbm.at[pl.ds(base + 80 * j, 80)], didx.at[j],
                               sem2) for j in range(4)]
    for p in dpends:
        p.wait()
    for i in range(nh):
        for j in range(4):
            pends.append(pltpu.async_copy(gemb_hbm.at[i].at[didx.at[j]],
                                          gam.at[i, pl.ds(80 * j, 80)], sem))
            pends.append(pltpu.async_copy(bemb_hbm.at[i].at[didx.at[j]],
                                          bet.at[i, pl.ds(80 * j, 80)], sem))
    for p in pends:
        p.wait()

    def blockf(t, carry):
        accs = list(carry)
        denvs = [denl[i, pl.ds(16 * t, 16)] + denl2[i, pl.ds(16 * t, 16)]
                 for i in range(nh)]
        for j in range(16):
            r = 16 * t + j
            for i in range(nh):
                nv = numl[r, pl.ds(16 * i, 16)] + numl2[r, pl.ds(16 * i, 16)]
                tt = nv / (denvs[i][j] + 1e-16)
                g = gam[i, r]
                b = bet[i, r]
                outl[r, pl.ds(16 * i, 16)] = g * tt + b
                accs[2 * i] = accs[2 * i] + g * g
                accs[2 * i + 1] = accs[2 * i + 1] + b * b
        return tuple(accs)
    accs = lax.fori_loop(0, _ROWS_W // 16, blockf, (zv,) * (2 * nh))
    for i in range(2 * nh):
        fbuf[i] = accs[i]
    pltpu.sync_copy(outl, out_hbm.at[pl.ds(base, _ROWS_W)])
    pltpu.sync_copy(fbuf, film_hbm.at[wid])


@functools.cache
def _node_call(nh):
    hw = nh * _H
    return pl.kernel(
        functools.partial(_node_body, nh),
        compiler_params=_SC_PARAMS,
        out_type=[jax.ShapeDtypeStruct((_NPAD, hw), jnp.float32),
                  jax.ShapeDtypeStruct((_NW, 2 * nh, 16), jnp.float32)],
        mesh=_sc_mesh(),
        scratch_types=[
            pltpu.VMEM((_ROWS_W, hw), jnp.float32),
            pltpu.VMEM((_ROWS_W, hw), jnp.float32),
            pltpu.VMEM((nh, _ROWS_W), jnp.float32),
            pltpu.VMEM((nh, _ROWS_W), jnp.float32),
            pltpu.VMEM((4, 80), jnp.int32),
            pltpu.VMEM((nh, _ROWS_W, _H), jnp.float32),
            pltpu.VMEM((nh, _ROWS_W, _H), jnp.float32),
            pltpu.VMEM((_ROWS_W, hw), jnp.float32),
            pltpu.VMEM((2 * nh, 16), jnp.float32),
            pltpu.SemaphoreType.DMA,
            pltpu.SemaphoreType.DMA,
        ],
    )


def _bias_body(idx_hbm, xc_hbm, o2_hbm, parts_hbm, idxl, rows48, rows16,
               fbuf, sem):
    cid = lax.axis_index("c")
    sid = lax.axis_index("s")
    wid = cid * 16 + sid
    zv = jnp.zeros((16,), jnp.float32)
    pltpu.sync_copy(idx_hbm.at[wid], idxl)
    pltpu.async_copy(xc_hbm.at[idxl], rows48, sem).wait()
    pltpu.async_copy(o2_hbm.at[idxl], rows16, sem).wait()

    def f(r, accs):
        a0, a1, a2, a3 = accs
        v0 = rows48[r, pl.ds(0, 16)]
        v1 = rows48[r, pl.ds(16, 16)]
        v2 = rows48[r, pl.ds(32, 16)]
        v3 = rows16[r]
        return (a0 + v0 * v0, a1 + v1 * v1, a2 + v2 * v2, a3 + v3 * v3)
    accs = lax.fori_loop(0, 32, f, (zv, zv, zv, zv))
    for i in range(4):
        fbuf[i] = accs[i]
    pltpu.sync_copy(fbuf, parts_hbm.at[wid])


@functools.cache
def _bias_call():
    return pl.kernel(
        _bias_body,
        compiler_params=_SC_PARAMS,
        out_type=jax.ShapeDtypeStruct((_NW, 4, 16), jnp.float32),
        mesh=_sc_mesh(),
        scratch_types=[
            pltpu.VMEM((32,), jnp.int32),
            pltpu.VMEM((32, 3 * _H), jnp.float32),
            pltpu.VMEM((32, _H), jnp.float32),
            pltpu.VMEM((4, 16), jnp.float32),
            pltpu.SemaphoreType.DMA,
        ],
    )


def kernel(x, adj, d, idx, edge, W_heads, a1_heads, a2_heads, gemb_heads,
           bemb_heads, W_out, a1_out, a2_out, gemb_out, bemb_out):
    x_pad = jnp.pad(x, ((0, _NPAD - _N), (0, 0)))
    src_p = jnp.pad(edge[0].astype(jnp.int32),
                    (0, _EPAD - _E)).reshape(_EPAD // 128, 128)
    dst_p = jnp.pad(edge[1].astype(jnp.int32), (0, _EPAD - _E),
                    constant_values=_NPAD - 1).reshape(_EPAD // 128, 128)
    adj_p = lax.bitcast_convert_type(
        jnp.pad(adj, (0, _EPAD - _E)), jnp.int32).reshape(_EPAD // 128, 128)
    eidx = jnp.pad(jnp.stack([src_p, dst_p, adj_p], axis=1),
                   ((0, 2), (0, 0), (0, 0)))
    zr = gemb_heads.shape[1]
    d_pad = jnp.pad(d.astype(jnp.int32), (0, _NPAD - _N), constant_values=zr)
    gemb_h = jnp.pad(gemb_heads, ((0, 0), (0, 1), (0, 0)))
    bemb_h = jnp.pad(bemb_heads, ((0, 0), (0, 1), (0, 0)))
    gemb_o = jnp.pad(gemb_out, ((0, 1), (0, 0)))[None]
    bemb_o = jnp.pad(bemb_out, ((0, 1), (0, 0)))[None]
    idx_p = jnp.pad(idx.astype(jnp.int32), (0, 1024 - 1000),
                    constant_values=_NPAD - 1).reshape(_NW, 32)

    hcat, s6 = _tc_feats(x_pad, W_heads, a1_heads, a2_heads)
    num1, den1 = _edge_call(3, _SPLIT0, 160 - _SPLIT0)(hcat, s6.T, eidx)
    xcat, film1 = _node_call(3)(num1, den1, d_pad, gemb_h, bemb_h)

    h2, s2 = _tc_feats(xcat, W_out[None], a1_out[None], a2_out[None])
    num2, den2 = _edge_call(1, _SPLIT0, 160 - _SPLIT0)(h2, s2.T, eidx)
    o2, film2 = _node_call(1)(num2, den2, d_pad, gemb_o, bemb_o)

    logp = _tc_log_softmax(o2)[:_N]
    parts = _bias_call()(idx_p, xcat, o2)

    scale_f = 1.0 / (_N * _H)
    f1 = jnp.sum(film1) * (scale_f / 3.0)
    f2 = jnp.sum(film2) * scale_f
    scale_b = 1.0 / (1000 * _H)
    b1 = jnp.sum(parts[:, :3]) * (scale_b / 3.0)
    b2 = jnp.sum(parts[:, 3]) * scale_b
    return (logp, b1 + b2, f1 + f2)
